# baseline (speedup 1.0000x reference)
import jax
import jax.numpy as jnp
from jax.experimental import pallas as pl

_B, _N, _M = 16, 2048, 64
_LANES = 128
_NR = _N * _M // _LANES
_R = 512
_NC = _NR // _R
_APC = _R * _LANES // _M


def _body(dist_ref, pt_ref, types_ref, coef_ref, off_ref, out_ref):
    j = pl.program_id(1)
    d = dist_ref[0]
    pt = pt_ref[0]
    is1 = pt == 1
    is2 = pt == 2

    def sel(i):
        return jnp.where(is1, coef_ref[i, 1],
                         jnp.where(is2, coef_ref[i, 2], coef_ref[i, 0]))

    a_ = sel(0)
    p_ = sel(1)
    u1 = sel(2)
    xisq = sel(3)
    v0 = sel(4)
    v1 = sel(5)
    ca = sel(6)
    inv_ba = sel(7)

    x = jnp.clip((d - ca) * inv_ba, 0.0, 1.0)
    x3 = x * x * x
    fc = 1.0 - x3 * (x * (6.0 * x - 15.0) + 10.0)

    phi = a_ * jnp.exp(p_ - u1 * d) * fc
    rho_e = xisq * jnp.exp(v0 - v1 * d) * fc

    phi_sum = jnp.sum(phi)

    lane = jax.lax.broadcasted_iota(jnp.int32, (_R, _LANES), 1)
    low = lane < _M
    s0 = jnp.sum(jnp.where(low, rho_e, 0.0), axis=1, keepdims=True)
    s_tot = jnp.sum(rho_e, axis=1, keepdims=True)
    s1 = s_tot - s0
    emb_sum = -jnp.sum(jnp.sqrt(s0 + 1e-12) + jnp.sqrt(s1 + 1e-12))

    tt = types_ref[0, 0]
    off_sum = jnp.sum(jnp.where(tt == 1, off_ref[0, 1], off_ref[0, 0]))

    e = jnp.reshape(0.5 * phi_sum + emb_sum + off_sum, (1, 1))

    @pl.when(j == 0)
    def _init():
        out_ref[0] = e

    @pl.when(j != 0)
    def _acc():
        out_ref[0] += e


def kernel(types, pair_types, distances, A, xi, p, q, r0, offset, cut_a, cut_b):
    dist = distances.reshape(_B, _NR, _LANES)
    pt = pair_types.reshape(_B, _NR, _LANES)
    types4 = types.reshape(_B, _NC, 1, _APC)

    coef = jnp.stack([
        A,
        p,
        p / r0,
        xi * xi,
        2.0 * q,
        2.0 * q / r0,
        cut_a,
        1.0 / (cut_b - cut_a),
    ])
    off2 = offset.reshape(1, 2)

    grid = (_B, _NC)
    energy = pl.pallas_call(
        _body,
        grid=grid,
        in_specs=[
            pl.BlockSpec((1, _R, _LANES), lambda b, j: (b, j, 0)),
            pl.BlockSpec((1, _R, _LANES), lambda b, j: (b, j, 0)),
            pl.BlockSpec((1, 1, 1, _APC), lambda b, j: (b, j, 0, 0)),
            pl.BlockSpec((8, 3), lambda b, j: (0, 0)),
            pl.BlockSpec((1, 2), lambda b, j: (0, 0)),
        ],
        out_specs=pl.BlockSpec((1, 1, 1), lambda b, j: (b, 0, 0)),
        out_shape=jax.ShapeDtypeStruct((_B, 1, 1), jnp.float32),
    )(dist, pt, types4, coef, off2)

    energy = energy.reshape(_B, 1)
    energy_per_atom = energy * (1.0 / _N)
    return (energy, energy_per_atom)


# speedup vs baseline: 1.3063x; 1.3063x over previous
import jax
import jax.numpy as jnp
from jax.experimental import pallas as pl

_B, _N, _M = 16, 2048, 64
_RA = 1024
_NC = _N // _RA


def _body(dist_ref, pt_ref, types_ref, coef_ref, out_ref):
    j = pl.program_id(1)
    d = dist_ref[0]
    pt = pt_ref[0]
    is1 = pt == 1
    is2 = pt == 2

    def sel(i):
        return jnp.where(is1, coef_ref[i, 1],
                         jnp.where(is2, coef_ref[i, 2], coef_ref[i, 0]))

    p0 = sel(0)
    p1 = sel(1)
    q0 = sel(2)
    q1 = sel(3)
    ga = sel(4)
    de = sel(5)

    x = jnp.clip(de * d - ga, 0.0, 1.0)
    x3 = x * x * x
    fc = 1.0 - x3 * (x * (6.0 * x - 15.0) + 10.0)

    half_phi = jnp.exp2(p0 - p1 * d) * fc
    rho_e = jnp.exp2(q0 - q1 * d) * fc

    half_phi_sum = jnp.sum(half_phi)
    s = jnp.sum(rho_e, axis=1, keepdims=True)
    emb_sum = jnp.sum(jnp.sqrt(s + 1e-12))

    tt = types_ref[0, 0]
    off_sum = jnp.sum(jnp.where(tt == 1, coef_ref[6, 1], coef_ref[6, 0]))

    e = jnp.reshape(half_phi_sum - emb_sum + off_sum, (1, 1))

    @pl.when(j == 0)
    def _init():
        out_ref[0] = e

    @pl.when(j != 0)
    def _acc():
        out_ref[0] += e


def kernel(types, pair_types, distances, A, xi, p, q, r0, offset, cut_a, cut_b):
    types4 = types.reshape(_B, _NC, 1, _RA)

    inv_ln2 = 1.4426950408889634
    inv_ba = 1.0 / (cut_b - cut_a)
    coef = jnp.concatenate([
        jnp.stack([
            jnp.log2(0.5 * A) + p * inv_ln2,
            (p / r0) * inv_ln2,
            2.0 * jnp.log2(xi) + 2.0 * q * inv_ln2,
            (2.0 * q / r0) * inv_ln2,
            cut_a * inv_ba,
            inv_ba,
        ]),
        jnp.pad(offset, (0, 1)).reshape(1, 3),
    ])

    energy = pl.pallas_call(
        _body,
        grid=(_B, _NC),
        in_specs=[
            pl.BlockSpec((1, _RA, _M), lambda b, j: (b, j, 0)),
            pl.BlockSpec((1, _RA, _M), lambda b, j: (b, j, 0)),
            pl.BlockSpec((1, 1, 1, _RA), lambda b, j: (b, j, 0, 0)),
            pl.BlockSpec((7, 3), lambda b, j: (0, 0)),
        ],
        out_specs=pl.BlockSpec((1, 1, 1), lambda b, j: (b, 0, 0)),
        out_shape=jax.ShapeDtypeStruct((_B, 1, 1), jnp.float32),
    )(distances, pair_types, types4, coef)

    energy = energy.reshape(_B, 1)
    energy_per_atom = energy * (1.0 / _N)
    return (energy, energy_per_atom)


# speedup vs baseline: 1.3190x; 1.0097x over previous
import jax
import jax.numpy as jnp
from jax.experimental import pallas as pl

_B, _N, _M = 16, 2048, 64
_RA = 1024
_NC = _N // _RA


def _body(dist_ref, pt_ref, types_ref, coef_ref, out_ref):
    j = pl.program_id(1)
    d = dist_ref[0]
    pt = pt_ref[0]
    is1 = pt == 1
    is2 = pt == 2

    def sel(i):
        return jnp.where(is1, coef_ref[i, 1],
                         jnp.where(is2, coef_ref[i, 2], coef_ref[i, 0]))

    p0 = sel(0)
    p1 = sel(1)
    q0 = sel(2)
    q1 = sel(3)
    ga = sel(4)
    de = sel(5)

    x = jnp.clip(de * d - ga, 0.0, 1.0)
    x3 = x * x * x
    fc = 1.0 - x3 * (x * (6.0 * x - 15.0) + 10.0)

    half_phi = jnp.exp2(p0 - p1 * d) * fc
    rho_e = jnp.exp2(q0 - q1 * d) * fc

    half_phi_sum = jnp.sum(half_phi)
    s = jnp.sum(rho_e, axis=1, keepdims=True) + 1e-12
    emb_sum = jnp.sum(s * jax.lax.rsqrt(s))

    tt = types_ref[0, 0]
    off_sum = jnp.sum(jnp.where(tt == 1, coef_ref[6, 1], coef_ref[6, 0]))

    e = jnp.reshape(half_phi_sum - emb_sum + off_sum, (1, 1))

    @pl.when(j == 0)
    def _init():
        out_ref[0] = e

    @pl.when(j != 0)
    def _acc():
        out_ref[0] += e


def kernel(types, pair_types, distances, A, xi, p, q, r0, offset, cut_a, cut_b):
    types4 = types.reshape(_B, _NC, 1, _RA)

    inv_ln2 = 1.4426950408889634
    inv_ba = 1.0 / (cut_b - cut_a)
    coef = jnp.concatenate([
        jnp.stack([
            jnp.log2(0.5 * A) + p * inv_ln2,
            (p / r0) * inv_ln2,
            2.0 * jnp.log2(xi) + 2.0 * q * inv_ln2,
            (2.0 * q / r0) * inv_ln2,
            cut_a * inv_ba,
            inv_ba,
        ]),
        jnp.pad(offset, (0, 1)).reshape(1, 3),
    ])

    energy = pl.pallas_call(
        _body,
        grid=(_B, _NC),
        in_specs=[
            pl.BlockSpec((1, _RA, _M), lambda b, j: (b, j, 0)),
            pl.BlockSpec((1, _RA, _M), lambda b, j: (b, j, 0)),
            pl.BlockSpec((1, 1, 1, _RA), lambda b, j: (b, j, 0, 0)),
            pl.BlockSpec((7, 3), lambda b, j: (0, 0)),
        ],
        out_specs=pl.BlockSpec((1, 1, 1), lambda b, j: (b, 0, 0)),
        out_shape=jax.ShapeDtypeStruct((_B, 1, 1), jnp.float32),
    )(distances, pair_types, types4, coef)

    energy = energy.reshape(_B, 1)
    energy_per_atom = energy * (1.0 / _N)
    return (energy, energy_per_atom)


# speedup vs baseline: 1.3328x; 1.0104x over previous
import jax
import jax.numpy as jnp
from jax.experimental import pallas as pl

_B, _N, _M = 16, 2048, 64
_RA = 1024
_NC = _N // _RA


def _body(dist_ref, pt_ref, types_ref, coef_ref, out_ref, epa_ref):
    b = pl.program_id(0)
    j = pl.program_id(1)
    d = dist_ref[0]
    pt = pt_ref[0]
    is1 = pt == 1
    is2 = pt == 2

    def sel(i):
        return jnp.where(is1, coef_ref[i, 1],
                         jnp.where(is2, coef_ref[i, 2], coef_ref[i, 0]))

    p0 = sel(0)
    p1 = sel(1)
    q0 = sel(2)
    q1 = sel(3)
    ga = sel(4)
    de = sel(5)

    x = jnp.clip(de * d - ga, 0.0, 1.0)
    x3 = x * x * x
    fc = 1.0 - x3 * (x * (6.0 * x - 15.0) + 10.0)

    half_phi = jnp.exp2(p0 - p1 * d) * fc
    rho_e = jnp.exp2(q0 - q1 * d) * fc

    half_phi_sum = jnp.sum(half_phi)
    s = jnp.sum(rho_e, axis=1, keepdims=True) + 1e-12
    emb_sum = jnp.sum(s * jax.lax.rsqrt(s))

    e = jnp.reshape(half_phi_sum - emb_sum, (1, 1))

    @pl.when(j == 0)
    def _init():
        tt = types_ref[pl.ds(b, 1), :]
        off_sum = jnp.sum(jnp.where(tt == 1, coef_ref[6, 1], coef_ref[6, 0]))
        out_ref[pl.ds(b, 1), :] = e + off_sum

    @pl.when(j != 0)
    def _acc():
        out_ref[pl.ds(b, 1), :] += e

    @pl.when(j == _NC - 1)
    def _fin():
        epa_ref[pl.ds(b, 1), :] = out_ref[pl.ds(b, 1), :] * (1.0 / _N)


def kernel(types, pair_types, distances, A, xi, p, q, r0, offset, cut_a, cut_b):
    inv_ln2 = 1.4426950408889634
    inv_ba = 1.0 / (cut_b - cut_a)
    coef = jnp.concatenate([
        jnp.stack([
            jnp.log2(0.5 * A) + p * inv_ln2,
            (p / r0) * inv_ln2,
            2.0 * jnp.log2(xi) + 2.0 * q * inv_ln2,
            (2.0 * q / r0) * inv_ln2,
            cut_a * inv_ba,
            inv_ba,
        ]),
        jnp.pad(offset, (0, 1)).reshape(1, 3),
    ])

    energy, energy_per_atom = pl.pallas_call(
        _body,
        grid=(_B, _NC),
        in_specs=[
            pl.BlockSpec((1, _RA, _M), lambda b, j: (b, j, 0)),
            pl.BlockSpec((1, _RA, _M), lambda b, j: (b, j, 0)),
            pl.BlockSpec((_B, _N), lambda b, j: (0, 0)),
            pl.BlockSpec((7, 3), lambda b, j: (0, 0)),
        ],
        out_specs=[
            pl.BlockSpec((_B, 1), lambda b, j: (0, 0)),
            pl.BlockSpec((_B, 1), lambda b, j: (0, 0)),
        ],
        out_shape=[
            jax.ShapeDtypeStruct((_B, 1), jnp.float32),
            jax.ShapeDtypeStruct((_B, 1), jnp.float32),
        ],
    )(distances, pair_types, types, coef)

    return (energy, energy_per_atom)
